# Initial kernel scaffold; baseline (speedup 1.0000x reference)
import jax
import jax.numpy as jnp
from jax.experimental import pallas as pl


def kernel(x, edge_index, batch, W1, b1, W2, b2, W3, b3, fc1_W, fc1_b, fc2a_W, fc2a_b, fc2_W, fc2_b, fc2b_W, fc2b_b, fc3_W, fc3_b):
    raise NotImplementedError("write your pallas kernel here")


# speedup vs baseline: 16.4375x; 16.4375x over previous
import functools

import jax
import jax.numpy as jnp
from jax import lax
from jax.experimental import pallas as pl
from jax.experimental.pallas import tpu as pltpu
from jax.experimental.pallas import tpu_sc as plsc

N = 10000
E = 320000
G = 64
D_IN = 128
DIM = 32

NC = 2
NS = 16
NW = NC * NS
C = 128
J = 79
E_PAD = NW * J * C
NACC = 10112
ZR = NACC // NS
DEGW = 8

_mesh = plsc.VectorSubcoreMesh(core_axis_name="c", subcore_axis_name="s")
_sc_params = pltpu.CompilerParams(use_tc_tiling_on_sc=False)


@functools.partial(
    pl.kernel,
    out_type=jax.ShapeDtypeStruct((NC, NACC, DEGW), jnp.float32),
    mesh=_mesh,
    scratch_types=[
        pltpu.VMEM((C,), jnp.int32),
        pltpu.VMEM((C, DEGW), jnp.float32),
        pltpu.VMEM_SHARED((NACC, DEGW), jnp.float32),
    ],
    compiler_params=_sc_params,
)
def _sc_degree(dst_hbm, ones_hbm, zeros_hbm, out_hbm, dst_v, ones_v, acc):
    cid = lax.axis_index("c")
    sid = lax.axis_index("s")
    wid = sid * NC + cid
    pltpu.sync_copy(ones_hbm, ones_v)
    pltpu.sync_copy(zeros_hbm.at[pl.ds(sid * ZR, ZR)], acc.at[pl.ds(sid * ZR, ZR)])
    plsc.subcore_barrier()

    def body(j, carry):
        pltpu.sync_copy(dst_hbm.at[wid, j], dst_v)
        pltpu.sync_copy(ones_v, acc.at[dst_v], add=True)
        return carry

    lax.fori_loop(0, J, body, 0)
    plsc.subcore_barrier()
    pltpu.sync_copy(acc.at[pl.ds(sid * ZR, ZR)], out_hbm.at[cid, pl.ds(sid * ZR, ZR)])


@functools.partial(
    pl.kernel,
    out_type=jax.ShapeDtypeStruct((NC, NACC, DIM), jnp.float32),
    mesh=_mesh,
    scratch_types=[
        pltpu.VMEM((C,), jnp.int32),
        pltpu.VMEM((C,), jnp.int32),
        pltpu.VMEM((C, DIM), jnp.float32),
        pltpu.VMEM_SHARED((NACC, DIM), jnp.float32),
        pltpu.SemaphoreType.DMA,
    ],
    compiler_params=_sc_params,
)
def _sc_edge_agg(src_hbm, dst_hbm, hp_hbm, zeros_hbm, out_hbm,
                 src_v, dst_v, rows_v, acc, sem):
    cid = lax.axis_index("c")
    sid = lax.axis_index("s")
    wid = sid * NC + cid
    pltpu.sync_copy(zeros_hbm.at[pl.ds(sid * ZR, ZR)], acc.at[pl.ds(sid * ZR, ZR)])
    plsc.subcore_barrier()

    def body(j, carry):
        pltpu.sync_copy(src_hbm.at[wid, j], src_v)
        pltpu.sync_copy(dst_hbm.at[wid, j], dst_v)
        pltpu.async_copy(hp_hbm.at[src_v], rows_v, sem).wait()
        pltpu.sync_copy(rows_v, acc.at[dst_v], add=True)
        return carry

    lax.fori_loop(0, J, body, 0)
    plsc.subcore_barrier()
    pltpu.sync_copy(acc.at[pl.ds(sid * ZR, ZR)], out_hbm.at[cid, pl.ds(sid * ZR, ZR)])


_R = 1000
_GRID = N // _R


def _prep_body(deg_ref, x_ref, w1_ref, hp_ref, dinv_ref):
    deg = deg_ref[0, :, 0:1] + deg_ref[1, :, 0:1] + 1.0
    dinv = lax.rsqrt(deg)
    h = jnp.dot(x_ref[...], w1_ref[...], preferred_element_type=jnp.float32)
    hp_ref[...] = h * dinv
    dinv_ref[...] = dinv


def _tc_prep(deg_parts, x, W1):
    return pl.pallas_call(
        _prep_body,
        grid=(_GRID,),
        in_specs=[
            pl.BlockSpec((NC, _R, DEGW), lambda i: (0, i, 0)),
            pl.BlockSpec((_R, D_IN), lambda i: (i, 0)),
            pl.BlockSpec((D_IN, DIM), lambda i: (0, 0)),
        ],
        out_specs=[
            pl.BlockSpec((_R, DIM), lambda i: (i, 0)),
            pl.BlockSpec((_R, 1), lambda i: (i, 0)),
        ],
        out_shape=[
            jax.ShapeDtypeStruct((N, DIM), jnp.float32),
            jax.ShapeDtypeStruct((N, 1), jnp.float32),
        ],
    )(deg_parts, x, W1)


def _mid_body(agg_ref, hp_ref, dinv_ref, b_ref, w_ref, x_ref, hpn_ref):
    dinv = dinv_ref[...]
    t = (agg_ref[0] + agg_ref[1] + hp_ref[...]) * dinv + b_ref[...]
    xi = jnp.maximum(t, 0.0)
    x_ref[...] = xi
    hpn_ref[...] = jnp.dot(xi, w_ref[...], preferred_element_type=jnp.float32) * dinv


def _tc_mid(agg_parts, hp, dinv, b, Wn):
    return pl.pallas_call(
        _mid_body,
        grid=(_GRID,),
        in_specs=[
            pl.BlockSpec((NC, _R, DIM), lambda i: (0, i, 0)),
            pl.BlockSpec((_R, DIM), lambda i: (i, 0)),
            pl.BlockSpec((_R, 1), lambda i: (i, 0)),
            pl.BlockSpec((1, DIM), lambda i: (0, 0)),
            pl.BlockSpec((DIM, DIM), lambda i: (0, 0)),
        ],
        out_specs=[
            pl.BlockSpec((_R, DIM), lambda i: (i, 0)),
            pl.BlockSpec((_R, DIM), lambda i: (i, 0)),
        ],
        out_shape=[
            jax.ShapeDtypeStruct((N, DIM), jnp.float32),
            jax.ShapeDtypeStruct((N, DIM), jnp.float32),
        ],
    )(agg_parts, hp, dinv, b.reshape(1, DIM), Wn)


def _head_body(agg_ref, hp3_ref, dinv_ref, b3_ref, x1_ref, x2_ref, batch_ref,
               fc1w_ref, fc1b_ref, fc2aw_ref, fc2ab_ref,
               fc2bw_ref, fc2bb_ref, fc3w_ref, fc3b_ref,
               out_ref, pooled_scr):
    i = pl.program_id(0)

    @pl.when(i == 0)
    def _zero():
        pooled_scr[...] = jnp.zeros_like(pooled_scr)

    dinv = dinv_ref[...]
    t = (agg_ref[0] + agg_ref[1] + hp3_ref[...]) * dinv + b3_ref[...]
    x3 = jnp.maximum(t, 0.0)
    h = (jnp.dot(x1_ref[...], fc1w_ref[0:DIM], preferred_element_type=jnp.float32)
         + jnp.dot(x2_ref[...], fc1w_ref[DIM:2 * DIM], preferred_element_type=jnp.float32)
         + jnp.dot(x3, fc1w_ref[2 * DIM:3 * DIM], preferred_element_type=jnp.float32)
         + fc1b_ref[...])
    h = jnp.maximum(h, 0.0)
    h2 = jnp.dot(h, fc2aw_ref[...], preferred_element_type=jnp.float32) + fc2ab_ref[...]
    h2 = jnp.maximum(h2, 0.0)
    gids = lax.broadcasted_iota(jnp.int32, (G, _R), 0)
    onehot = (gids == batch_ref[0]).astype(jnp.float32)
    pooled_scr[...] += jnp.dot(onehot, h2, preferred_element_type=jnp.float32)

    @pl.when(i == pl.num_programs(0) - 1)
    def _head():
        ph = pooled_scr[...]
        hb = jnp.dot(ph, fc2bw_ref[...], preferred_element_type=jnp.float32) + fc2bb_ref[...]
        hb = jnp.maximum(hb, 0.0)
        lg = jnp.dot(hb, fc3w_ref[...], preferred_element_type=jnp.float32) + fc3b_ref[...]
        m = jnp.max(lg, axis=-1, keepdims=True)
        s = jnp.sum(jnp.exp(lg - m), axis=-1, keepdims=True)
        out_ref[...] = (lg - m) - jnp.log(s)


def _tc_head(agg_parts, hp3, dinv, b3, x1, x2, batch2d,
             fc1_W, fc1_b, fc2a_W, fc2a_b, fc2b_W, fc2b_b, fc3_W, fc3_b):
    OUT = fc3_W.shape[1]
    GD2 = fc2b_W.shape[1]
    GD = fc2a_W.shape[1]
    FD = fc1_W.shape[1]
    return pl.pallas_call(
        _head_body,
        grid=(_GRID,),
        in_specs=[
            pl.BlockSpec((NC, _R, DIM), lambda i: (0, i, 0)),
            pl.BlockSpec((_R, DIM), lambda i: (i, 0)),
            pl.BlockSpec((_R, 1), lambda i: (i, 0)),
            pl.BlockSpec((1, DIM), lambda i: (0, 0)),
            pl.BlockSpec((_R, DIM), lambda i: (i, 0)),
            pl.BlockSpec((_R, DIM), lambda i: (i, 0)),
            pl.BlockSpec((1, 1, _R), lambda i: (i, 0, 0)),
            pl.BlockSpec((3 * DIM, FD), lambda i: (0, 0)),
            pl.BlockSpec((1, FD), lambda i: (0, 0)),
            pl.BlockSpec((FD, GD), lambda i: (0, 0)),
            pl.BlockSpec((1, GD), lambda i: (0, 0)),
            pl.BlockSpec((GD, GD2), lambda i: (0, 0)),
            pl.BlockSpec((1, GD2), lambda i: (0, 0)),
            pl.BlockSpec((GD2, OUT), lambda i: (0, 0)),
            pl.BlockSpec((1, OUT), lambda i: (0, 0)),
        ],
        out_specs=pl.BlockSpec((G, OUT), lambda i: (0, 0)),
        out_shape=jax.ShapeDtypeStruct((G, OUT), jnp.float32),
        scratch_shapes=[pltpu.VMEM((G, GD), jnp.float32)],
    )(agg_parts, hp3, dinv, b3.reshape(1, DIM), x1, x2, batch2d,
      fc1_W, fc1_b.reshape(1, FD), fc2a_W, fc2a_b.reshape(1, GD),
      fc2b_W, fc2b_b.reshape(1, GD2), fc3_W, fc3_b.reshape(1, OUT))


def kernel(x, edge_index, batch, W1, b1, W2, b2, W3, b3,
           fc1_W, fc1_b, fc2a_W, fc2a_b, fc2_W, fc2_b,
           fc2b_W, fc2b_b, fc3_W, fc3_b):
    del fc2_W, fc2_b

    pad = E_PAD - E
    src = jnp.concatenate([edge_index[0], jnp.zeros((pad,), jnp.int32)])
    dst = jnp.concatenate([edge_index[1], jnp.full((pad,), N, jnp.int32)])
    src = src.reshape(NW, J, C)
    dst = dst.reshape(NW, J, C)

    ones_deg = jnp.ones((C, DEGW), jnp.float32)
    zeros_deg = jnp.zeros((NACC, DEGW), jnp.float32)
    zeros_acc = jnp.zeros((NACC, DIM), jnp.float32)

    deg_parts = _sc_degree(dst, ones_deg, zeros_deg)
    hp1, dinv = _tc_prep(deg_parts, x, W1)

    agg1 = _sc_edge_agg(src, dst, hp1, zeros_acc)
    x1, hp2 = _tc_mid(agg1, hp1, dinv, b1, W2)

    agg2 = _sc_edge_agg(src, dst, hp2, zeros_acc)
    x2, hp3 = _tc_mid(agg2, hp2, dinv, b2, W3)

    agg3 = _sc_edge_agg(src, dst, hp3, zeros_acc)

    batch2d = batch.reshape(_GRID, 1, _R)
    return _tc_head(agg3, hp3, dinv, b3, x1, x2, batch2d,
                    fc1_W, fc1_b, fc2a_W, fc2a_b, fc2b_W, fc2b_b, fc3_W, fc3_b)


# speedup vs baseline: 24.1468x; 1.4690x over previous
import functools

import jax
import jax.numpy as jnp
from jax import lax
from jax.experimental import pallas as pl
from jax.experimental.pallas import tpu as pltpu
from jax.experimental.pallas import tpu_sc as plsc

N = 10000
E = 320000
G = 64
D_IN = 128
DIM = 32

NC = 2
NS = 16
NW = NC * NS
C = 128
J = 80
E_PAD = NW * J * C
NACC = 10112
ZR = NACC // NS
DEGW = 8
NBUF = 4

_mesh = plsc.VectorSubcoreMesh(core_axis_name="c", subcore_axis_name="s")
_sc_params = pltpu.CompilerParams(use_tc_tiling_on_sc=False)


@functools.partial(
    pl.kernel,
    out_type=jax.ShapeDtypeStruct((NC, NACC, DEGW), jnp.float32),
    mesh=_mesh,
    scratch_types=(
        [pltpu.VMEM((C,), jnp.int32)] * 4
        + [pltpu.VMEM((C, DEGW), jnp.float32),
           pltpu.VMEM_SHARED((NACC, DEGW), jnp.float32)]
        + [pltpu.SemaphoreType.DMA] * 8
    ),
    compiler_params=_sc_params,
)
def _sc_degree(dst_hbm, ones_hbm, zeros_hbm, out_hbm,
               d0, d1, d2, d3, ones_v, acc, *sems):
    didx = [d0, d1, d2, d3]
    isem = sems[:4]
    ssem = sems[4:]
    cid = lax.axis_index("c")
    sid = lax.axis_index("s")
    wid = sid * NC + cid
    pltpu.sync_copy(ones_hbm, ones_v)
    pltpu.sync_copy(zeros_hbm.at[pl.ds(sid * ZR, ZR)], acc.at[pl.ds(sid * ZR, ZR)])
    plsc.subcore_barrier()

    def F(j, m):
        pltpu.async_copy(dst_hbm.at[wid, j], didx[m], isem[m])

    def Fw(j, m):
        pltpu.make_async_copy(dst_hbm.at[wid, j], didx[m], isem[m]).wait()

    def S(j, m):
        pltpu.sync_copy(ones_v, acc.at[didx[m]], add=True)

    F(0, 0)
    F(1, 1)

    def body(jj, carry):
        for b in range(4):
            j = jj * 4 + b
            Fw(j, b)
            S(j, b)

            @pl.when(j + 2 < J)
            def _prefetch():
                F(j + 2, (b + 2) % 4)

        return carry

    lax.fori_loop(0, J // 4, body, 0)
    plsc.subcore_barrier()
    pltpu.sync_copy(acc.at[pl.ds(sid * ZR, ZR)], out_hbm.at[cid, pl.ds(sid * ZR, ZR)])


@functools.partial(
    pl.kernel,
    out_type=jax.ShapeDtypeStruct((NC, NACC, DIM), jnp.float32),
    mesh=_mesh,
    scratch_types=(
        [pltpu.VMEM((C,), jnp.int32)] * 16
        + [pltpu.VMEM((C, DIM), jnp.float32)] * 4
        + [pltpu.VMEM_SHARED((NACC, DIM), jnp.float32)]
        + [pltpu.SemaphoreType.DMA] * 16
    ),
    compiler_params=_sc_params,
)
def _sc_edge_agg(src_hbm, dst_hbm, hp_hbm, zeros_hbm, out_hbm, *scr):
    sidx = scr[0:8]
    didx = scr[8:16]
    rows = scr[16:20]
    acc = scr[20]
    isem = scr[21:29]
    gsem = scr[29:33]
    ssem = scr[33:37]
    cid = lax.axis_index("c")
    sid = lax.axis_index("s")
    wid = sid * NC + cid
    pltpu.sync_copy(zeros_hbm.at[pl.ds(sid * ZR, ZR)], acc.at[pl.ds(sid * ZR, ZR)])
    plsc.subcore_barrier()

    def F(j, m):
        pltpu.async_copy(src_hbm.at[wid, j], sidx[m], isem[m])
        pltpu.async_copy(dst_hbm.at[wid, j], didx[m], isem[m])

    def Fw(j, m):
        pltpu.make_async_copy(src_hbm.at[wid, j], sidx[m], isem[m]).wait()
        pltpu.make_async_copy(dst_hbm.at[wid, j], didx[m], isem[m]).wait()

    def G(j, m, r):
        pltpu.async_copy(hp_hbm.at[sidx[m]], rows[r], gsem[r])

    def Gw(j, m, r):
        pltpu.make_async_copy(hp_hbm.at[sidx[m]], rows[r], gsem[r]).wait()

    def S(j, m, r):
        pltpu.sync_copy(rows[r], acc.at[didx[m]], add=True)

    for m in range(4):
        F(m, m)
    Fw(0, 0)
    G(0, 0, 0)
    Fw(1, 1)
    G(1, 1, 1)

    def body(jj, carry):
        for b in range(8):
            j = jj * 8 + b
            rb = b % 4
            Gw(j, b, rb)

            @pl.when(j + 2 < J)
            def _gather_ahead():
                Fw(j + 2, (b + 2) % 8)
                G(j + 2, (b + 2) % 8, (rb + 2) % 4)

            @pl.when(j + 4 < J)
            def _fetch_ahead():
                F(j + 4, (b + 4) % 8)

            S(j, b, rb)

        return carry

    lax.fori_loop(0, J // 8, body, 0)
    plsc.subcore_barrier()
    pltpu.sync_copy(acc.at[pl.ds(sid * ZR, ZR)], out_hbm.at[cid, pl.ds(sid * ZR, ZR)])


_R = 1000
_GRID = N // _R


def _mm1_body(x_ref, w1_ref, h_ref):
    h_ref[...] = jnp.dot(x_ref[...], w1_ref[...], preferred_element_type=jnp.float32)


def _tc_mm1(x, W1):
    return pl.pallas_call(
        _mm1_body,
        grid=(_GRID,),
        in_specs=[
            pl.BlockSpec((_R, D_IN), lambda i: (i, 0)),
            pl.BlockSpec((D_IN, DIM), lambda i: (0, 0)),
        ],
        out_specs=pl.BlockSpec((_R, DIM), lambda i: (i, 0)),
        out_shape=jax.ShapeDtypeStruct((N, DIM), jnp.float32),
    )(x, W1)


def _scale_body(deg_ref, h_ref, hp_ref, dinv_ref):
    deg = deg_ref[0, :, 0:1] + deg_ref[1, :, 0:1] + 1.0
    dinv = lax.rsqrt(deg)
    hp_ref[...] = h_ref[...] * dinv
    dinv_ref[...] = dinv


def _tc_scale(deg_parts, h1):
    return pl.pallas_call(
        _scale_body,
        grid=(_GRID,),
        in_specs=[
            pl.BlockSpec((NC, _R, DEGW), lambda i: (0, i, 0)),
            pl.BlockSpec((_R, DIM), lambda i: (i, 0)),
        ],
        out_specs=[
            pl.BlockSpec((_R, DIM), lambda i: (i, 0)),
            pl.BlockSpec((_R, 1), lambda i: (i, 0)),
        ],
        out_shape=[
            jax.ShapeDtypeStruct((N, DIM), jnp.float32),
            jax.ShapeDtypeStruct((N, 1), jnp.float32),
        ],
    )(deg_parts, h1)


def _mid_body(agg_ref, hp_ref, dinv_ref, b_ref, w_ref, x_ref, hpn_ref):
    dinv = dinv_ref[...]
    t = (agg_ref[0] + agg_ref[1] + hp_ref[...]) * dinv + b_ref[...]
    xi = jnp.maximum(t, 0.0)
    x_ref[...] = xi
    hpn_ref[...] = jnp.dot(xi, w_ref[...], preferred_element_type=jnp.float32) * dinv


def _tc_mid(agg_parts, hp, dinv, b, Wn):
    return pl.pallas_call(
        _mid_body,
        grid=(_GRID,),
        in_specs=[
            pl.BlockSpec((NC, _R, DIM), lambda i: (0, i, 0)),
            pl.BlockSpec((_R, DIM), lambda i: (i, 0)),
            pl.BlockSpec((_R, 1), lambda i: (i, 0)),
            pl.BlockSpec((1, DIM), lambda i: (0, 0)),
            pl.BlockSpec((DIM, DIM), lambda i: (0, 0)),
        ],
        out_specs=[
            pl.BlockSpec((_R, DIM), lambda i: (i, 0)),
            pl.BlockSpec((_R, DIM), lambda i: (i, 0)),
        ],
        out_shape=[
            jax.ShapeDtypeStruct((N, DIM), jnp.float32),
            jax.ShapeDtypeStruct((N, DIM), jnp.float32),
        ],
    )(agg_parts, hp, dinv, b.reshape(1, DIM), Wn)


def _head_body(agg_ref, hp3_ref, dinv_ref, b3_ref, x1_ref, x2_ref, batch_ref,
               fc1w_ref, fc1b_ref, fc2aw_ref, fc2ab_ref,
               fc2bw_ref, fc2bb_ref, fc3w_ref, fc3b_ref,
               out_ref, pooled_scr):
    i = pl.program_id(0)

    @pl.when(i == 0)
    def _zero():
        pooled_scr[...] = jnp.zeros_like(pooled_scr)

    dinv = dinv_ref[...]
    t = (agg_ref[0] + agg_ref[1] + hp3_ref[...]) * dinv + b3_ref[...]
    x3 = jnp.maximum(t, 0.0)
    h = (jnp.dot(x1_ref[...], fc1w_ref[0:DIM], preferred_element_type=jnp.float32)
         + jnp.dot(x2_ref[...], fc1w_ref[DIM:2 * DIM], preferred_element_type=jnp.float32)
         + jnp.dot(x3, fc1w_ref[2 * DIM:3 * DIM], preferred_element_type=jnp.float32)
         + fc1b_ref[...])
    h = jnp.maximum(h, 0.0)
    h2 = jnp.dot(h, fc2aw_ref[...], preferred_element_type=jnp.float32) + fc2ab_ref[...]
    h2 = jnp.maximum(h2, 0.0)
    gids = lax.broadcasted_iota(jnp.int32, (G, _R), 0)
    onehot = (gids == batch_ref[0]).astype(jnp.float32)
    pooled_scr[...] += jnp.dot(onehot, h2, preferred_element_type=jnp.float32)

    @pl.when(i == pl.num_programs(0) - 1)
    def _head():
        ph = pooled_scr[...]
        hb = jnp.dot(ph, fc2bw_ref[...], preferred_element_type=jnp.float32) + fc2bb_ref[...]
        hb = jnp.maximum(hb, 0.0)
        lg = jnp.dot(hb, fc3w_ref[...], preferred_element_type=jnp.float32) + fc3b_ref[...]
        m = jnp.max(lg, axis=-1, keepdims=True)
        s = jnp.sum(jnp.exp(lg - m), axis=-1, keepdims=True)
        out_ref[...] = (lg - m) - jnp.log(s)


def _tc_head(agg_parts, hp3, dinv, b3, x1, x2, batch3d,
             fc1_W, fc1_b, fc2a_W, fc2a_b, fc2b_W, fc2b_b, fc3_W, fc3_b):
    OUT = fc3_W.shape[1]
    GD2 = fc2b_W.shape[1]
    GD = fc2a_W.shape[1]
    FD = fc1_W.shape[1]
    return pl.pallas_call(
        _head_body,
        grid=(_GRID,),
        in_specs=[
            pl.BlockSpec((NC, _R, DIM), lambda i: (0, i, 0)),
            pl.BlockSpec((_R, DIM), lambda i: (i, 0)),
            pl.BlockSpec((_R, 1), lambda i: (i, 0)),
            pl.BlockSpec((1, DIM), lambda i: (0, 0)),
            pl.BlockSpec((_R, DIM), lambda i: (i, 0)),
            pl.BlockSpec((_R, DIM), lambda i: (i, 0)),
            pl.BlockSpec((1, 1, _R), lambda i: (i, 0, 0)),
            pl.BlockSpec((3 * DIM, FD), lambda i: (0, 0)),
            pl.BlockSpec((1, FD), lambda i: (0, 0)),
            pl.BlockSpec((FD, GD), lambda i: (0, 0)),
            pl.BlockSpec((1, GD), lambda i: (0, 0)),
            pl.BlockSpec((GD, GD2), lambda i: (0, 0)),
            pl.BlockSpec((1, GD2), lambda i: (0, 0)),
            pl.BlockSpec((GD2, OUT), lambda i: (0, 0)),
            pl.BlockSpec((1, OUT), lambda i: (0, 0)),
        ],
        out_specs=pl.BlockSpec((G, OUT), lambda i: (0, 0)),
        out_shape=jax.ShapeDtypeStruct((G, OUT), jnp.float32),
        scratch_shapes=[pltpu.VMEM((G, GD), jnp.float32)],
    )(agg_parts, hp3, dinv, b3.reshape(1, DIM), x1, x2, batch3d,
      fc1_W, fc1_b.reshape(1, FD), fc2a_W, fc2a_b.reshape(1, GD),
      fc2b_W, fc2b_b.reshape(1, GD2), fc3_W, fc3_b.reshape(1, OUT))


def kernel(x, edge_index, batch, W1, b1, W2, b2, W3, b3,
           fc1_W, fc1_b, fc2a_W, fc2a_b, fc2_W, fc2_b,
           fc2b_W, fc2b_b, fc3_W, fc3_b):
    del fc2_W, fc2_b

    pad = E_PAD - E
    src = jnp.concatenate([edge_index[0], jnp.zeros((pad,), jnp.int32)]).reshape(NW, J, C)
    dst = jnp.concatenate([edge_index[1], jnp.full((pad,), N, jnp.int32)]).reshape(NW, J, C)

    ones_deg = jnp.ones((C, DEGW), jnp.float32)
    zeros_deg = jnp.zeros((NACC, DEGW), jnp.float32)
    zeros_acc = jnp.zeros((NACC, DIM), jnp.float32)

    h1 = _tc_mm1(x, W1)
    deg_parts = _sc_degree(dst, ones_deg, zeros_deg)
    hp1, dinv = _tc_scale(deg_parts, h1)

    agg1 = _sc_edge_agg(src, dst, hp1, zeros_acc)
    x1, hp2 = _tc_mid(agg1, hp1, dinv, b1, W2)

    agg2 = _sc_edge_agg(src, dst, hp2, zeros_acc)
    x2, hp3 = _tc_mid(agg2, hp2, dinv, b2, W3)

    agg3 = _sc_edge_agg(src, dst, hp3, zeros_acc)

    batch3d = batch.reshape(_GRID, 1, _R)
    return _tc_head(agg3, hp3, dinv, b3, x1, x2, batch3d,
                    fc1_W, fc1_b, fc2a_W, fc2a_b, fc2b_W, fc2b_b, fc3_W, fc3_b)


# speedup vs baseline: 24.2967x; 1.0062x over previous
import functools

import jax
import jax.numpy as jnp
from jax import lax
from jax.experimental import pallas as pl
from jax.experimental.pallas import tpu as pltpu
from jax.experimental.pallas import tpu_sc as plsc

N = 10000
E = 320000
G = 64
D_IN = 128
DIM = 32

NC = 2
NS = 16
NW = NC * NS
C = 128
J = 80
E_PAD = NW * J * C
NACC = 10112
ZR = NACC // NS
DEGW = 8
NBUF = 4

_mesh = plsc.VectorSubcoreMesh(core_axis_name="c", subcore_axis_name="s")
_sc_params = pltpu.CompilerParams(use_tc_tiling_on_sc=False)


@functools.partial(
    pl.kernel,
    out_type=jax.ShapeDtypeStruct((NC, NACC, DEGW), jnp.float32),
    mesh=_mesh,
    scratch_types=(
        [pltpu.VMEM((C,), jnp.int32)] * 4
        + [pltpu.VMEM((C, DEGW), jnp.float32),
           pltpu.VMEM_SHARED((NACC, DEGW), jnp.float32)]
        + [pltpu.SemaphoreType.DMA] * 8
    ),
    compiler_params=_sc_params,
)
def _sc_degree(dst_hbm, ones_hbm, zeros_hbm, out_hbm,
               d0, d1, d2, d3, ones_v, acc, *sems):
    didx = [d0, d1, d2, d3]
    isem = sems[:4]
    ssem = sems[4:]
    cid = lax.axis_index("c")
    sid = lax.axis_index("s")
    wid = sid * NC + cid
    pltpu.sync_copy(ones_hbm, ones_v)
    pltpu.sync_copy(zeros_hbm.at[pl.ds(sid * ZR, ZR)], acc.at[pl.ds(sid * ZR, ZR)])
    plsc.subcore_barrier()

    def F(j, m):
        pltpu.async_copy(dst_hbm.at[wid, j], didx[m], isem[m])

    def Fw(j, m):
        pltpu.make_async_copy(dst_hbm.at[wid, j], didx[m], isem[m]).wait()

    def S(j, m):
        pltpu.async_copy(ones_v, acc.at[didx[m]], ssem[m], add=True)

    def Sw(j, m):
        pltpu.make_async_copy(ones_v, acc.at[didx[m]], ssem[m]).wait()

    F(0, 0)
    F(1, 1)

    def body(jj, carry):
        for b in range(4):
            j = jj * 4 + b
            Fw(j, b)
            S(j, b)

            @pl.when(j >= 2)
            def _drain():
                Sw(j - 2, (b + 2) % 4)

            @pl.when(j + 2 < J)
            def _prefetch():
                F(j + 2, (b + 2) % 4)

        return carry

    lax.fori_loop(0, J // 4, body, 0)
    Sw(J - 2, (J - 2) % 4)
    Sw(J - 1, (J - 1) % 4)
    plsc.subcore_barrier()
    pltpu.sync_copy(acc.at[pl.ds(sid * ZR, ZR)], out_hbm.at[cid, pl.ds(sid * ZR, ZR)])


@functools.partial(
    pl.kernel,
    out_type=jax.ShapeDtypeStruct((NC, NACC, DIM), jnp.float32),
    mesh=_mesh,
    scratch_types=(
        [pltpu.VMEM((C,), jnp.int32)] * 16
        + [pltpu.VMEM((C, DIM), jnp.float32)] * 4
        + [pltpu.VMEM_SHARED((NACC, DIM), jnp.float32)]
        + [pltpu.SemaphoreType.DMA] * 16
    ),
    compiler_params=_sc_params,
)
def _sc_edge_agg(src_hbm, dst_hbm, hp_hbm, zeros_hbm, out_hbm, *scr):
    sidx = scr[0:8]
    didx = scr[8:16]
    rows = scr[16:20]
    acc = scr[20]
    isem = scr[21:29]
    gsem = scr[29:33]
    ssem = scr[33:37]
    cid = lax.axis_index("c")
    sid = lax.axis_index("s")
    wid = sid * NC + cid
    pltpu.sync_copy(zeros_hbm.at[pl.ds(sid * ZR, ZR)], acc.at[pl.ds(sid * ZR, ZR)])
    plsc.subcore_barrier()

    def F(j, m):
        pltpu.async_copy(src_hbm.at[wid, j], sidx[m], isem[m])
        pltpu.async_copy(dst_hbm.at[wid, j], didx[m], isem[m])

    def Fw(j, m):
        pltpu.make_async_copy(src_hbm.at[wid, j], sidx[m], isem[m]).wait()
        pltpu.make_async_copy(dst_hbm.at[wid, j], didx[m], isem[m]).wait()

    def G(j, m, r):
        pltpu.async_copy(hp_hbm.at[sidx[m]], rows[r], gsem[r])

    def Gw(j, m, r):
        pltpu.make_async_copy(hp_hbm.at[sidx[m]], rows[r], gsem[r]).wait()

    def S(j, m, r):
        pltpu.async_copy(rows[r], acc.at[didx[m]], ssem[r], add=True)

    def Sw(j, m, r):
        pltpu.make_async_copy(rows[r], acc.at[didx[m]], ssem[r]).wait()

    for m in range(4):
        F(m, m)
    Fw(0, 0)
    G(0, 0, 0)
    Fw(1, 1)
    G(1, 1, 1)

    def body(jj, carry):
        for b in range(8):
            j = jj * 8 + b
            rb = b % 4
            Gw(j, b, rb)
            S(j, b, rb)

            @pl.when(j >= 2)
            def _drain():
                Sw(j - 2, (b + 6) % 8, (rb + 2) % 4)

            @pl.when(j + 2 < J)
            def _gather_ahead():
                Fw(j + 2, (b + 2) % 8)
                G(j + 2, (b + 2) % 8, (rb + 2) % 4)

            @pl.when(j + 4 < J)
            def _fetch_ahead():
                F(j + 4, (b + 4) % 8)

        return carry

    lax.fori_loop(0, J // 8, body, 0)
    Sw(J - 2, (J - 2) % 8, (J - 2) % 4)
    Sw(J - 1, (J - 1) % 8, (J - 1) % 4)
    plsc.subcore_barrier()
    pltpu.sync_copy(acc.at[pl.ds(sid * ZR, ZR)], out_hbm.at[cid, pl.ds(sid * ZR, ZR)])


_R = 1000
_GRID = N // _R


def _mm1_body(x_ref, w1_ref, h_ref):
    h_ref[...] = jnp.dot(x_ref[...], w1_ref[...], preferred_element_type=jnp.float32)


def _tc_mm1(x, W1):
    return pl.pallas_call(
        _mm1_body,
        grid=(_GRID,),
        in_specs=[
            pl.BlockSpec((_R, D_IN), lambda i: (i, 0)),
            pl.BlockSpec((D_IN, DIM), lambda i: (0, 0)),
        ],
        out_specs=pl.BlockSpec((_R, DIM), lambda i: (i, 0)),
        out_shape=jax.ShapeDtypeStruct((N, DIM), jnp.float32),
    )(x, W1)


def _scale_body(deg_ref, h_ref, hp_ref, dinv_ref):
    deg = deg_ref[0, :, 0:1] + deg_ref[1, :, 0:1] + 1.0
    dinv = lax.rsqrt(deg)
    hp_ref[...] = h_ref[...] * dinv
    dinv_ref[...] = dinv


def _tc_scale(deg_parts, h1):
    return pl.pallas_call(
        _scale_body,
        grid=(_GRID,),
        in_specs=[
            pl.BlockSpec((NC, _R, DEGW), lambda i: (0, i, 0)),
            pl.BlockSpec((_R, DIM), lambda i: (i, 0)),
        ],
        out_specs=[
            pl.BlockSpec((_R, DIM), lambda i: (i, 0)),
            pl.BlockSpec((_R, 1), lambda i: (i, 0)),
        ],
        out_shape=[
            jax.ShapeDtypeStruct((N, DIM), jnp.float32),
            jax.ShapeDtypeStruct((N, 1), jnp.float32),
        ],
    )(deg_parts, h1)


def _mid_body(agg_ref, hp_ref, dinv_ref, b_ref, w_ref, x_ref, hpn_ref):
    dinv = dinv_ref[...]
    t = (agg_ref[0] + agg_ref[1] + hp_ref[...]) * dinv + b_ref[...]
    xi = jnp.maximum(t, 0.0)
    x_ref[...] = xi
    hpn_ref[...] = jnp.dot(xi, w_ref[...], preferred_element_type=jnp.float32) * dinv


def _tc_mid(agg_parts, hp, dinv, b, Wn):
    return pl.pallas_call(
        _mid_body,
        grid=(_GRID,),
        in_specs=[
            pl.BlockSpec((NC, _R, DIM), lambda i: (0, i, 0)),
            pl.BlockSpec((_R, DIM), lambda i: (i, 0)),
            pl.BlockSpec((_R, 1), lambda i: (i, 0)),
            pl.BlockSpec((1, DIM), lambda i: (0, 0)),
            pl.BlockSpec((DIM, DIM), lambda i: (0, 0)),
        ],
        out_specs=[
            pl.BlockSpec((_R, DIM), lambda i: (i, 0)),
            pl.BlockSpec((_R, DIM), lambda i: (i, 0)),
        ],
        out_shape=[
            jax.ShapeDtypeStruct((N, DIM), jnp.float32),
            jax.ShapeDtypeStruct((N, DIM), jnp.float32),
        ],
    )(agg_parts, hp, dinv, b.reshape(1, DIM), Wn)


def _head_body(agg_ref, hp3_ref, dinv_ref, b3_ref, x1_ref, x2_ref, batch_ref,
               fc1w_ref, fc1b_ref, fc2aw_ref, fc2ab_ref,
               fc2bw_ref, fc2bb_ref, fc3w_ref, fc3b_ref,
               out_ref, pooled_scr):
    i = pl.program_id(0)

    @pl.when(i == 0)
    def _zero():
        pooled_scr[...] = jnp.zeros_like(pooled_scr)

    dinv = dinv_ref[...]
    t = (agg_ref[0] + agg_ref[1] + hp3_ref[...]) * dinv + b3_ref[...]
    x3 = jnp.maximum(t, 0.0)
    h = (jnp.dot(x1_ref[...], fc1w_ref[0:DIM], preferred_element_type=jnp.float32)
         + jnp.dot(x2_ref[...], fc1w_ref[DIM:2 * DIM], preferred_element_type=jnp.float32)
         + jnp.dot(x3, fc1w_ref[2 * DIM:3 * DIM], preferred_element_type=jnp.float32)
         + fc1b_ref[...])
    h = jnp.maximum(h, 0.0)
    h2 = jnp.dot(h, fc2aw_ref[...], preferred_element_type=jnp.float32) + fc2ab_ref[...]
    h2 = jnp.maximum(h2, 0.0)
    gids = lax.broadcasted_iota(jnp.int32, (G, _R), 0)
    onehot = (gids == batch_ref[0]).astype(jnp.float32)
    pooled_scr[...] += jnp.dot(onehot, h2, preferred_element_type=jnp.float32)

    @pl.when(i == pl.num_programs(0) - 1)
    def _head():
        ph = pooled_scr[...]
        hb = jnp.dot(ph, fc2bw_ref[...], preferred_element_type=jnp.float32) + fc2bb_ref[...]
        hb = jnp.maximum(hb, 0.0)
        lg = jnp.dot(hb, fc3w_ref[...], preferred_element_type=jnp.float32) + fc3b_ref[...]
        m = jnp.max(lg, axis=-1, keepdims=True)
        s = jnp.sum(jnp.exp(lg - m), axis=-1, keepdims=True)
        out_ref[...] = (lg - m) - jnp.log(s)


def _tc_head(agg_parts, hp3, dinv, b3, x1, x2, batch3d,
             fc1_W, fc1_b, fc2a_W, fc2a_b, fc2b_W, fc2b_b, fc3_W, fc3_b):
    OUT = fc3_W.shape[1]
    GD2 = fc2b_W.shape[1]
    GD = fc2a_W.shape[1]
    FD = fc1_W.shape[1]
    return pl.pallas_call(
        _head_body,
        grid=(_GRID,),
        in_specs=[
            pl.BlockSpec((NC, _R, DIM), lambda i: (0, i, 0)),
            pl.BlockSpec((_R, DIM), lambda i: (i, 0)),
            pl.BlockSpec((_R, 1), lambda i: (i, 0)),
            pl.BlockSpec((1, DIM), lambda i: (0, 0)),
            pl.BlockSpec((_R, DIM), lambda i: (i, 0)),
            pl.BlockSpec((_R, DIM), lambda i: (i, 0)),
            pl.BlockSpec((1, 1, _R), lambda i: (i, 0, 0)),
            pl.BlockSpec((3 * DIM, FD), lambda i: (0, 0)),
            pl.BlockSpec((1, FD), lambda i: (0, 0)),
            pl.BlockSpec((FD, GD), lambda i: (0, 0)),
            pl.BlockSpec((1, GD), lambda i: (0, 0)),
            pl.BlockSpec((GD, GD2), lambda i: (0, 0)),
            pl.BlockSpec((1, GD2), lambda i: (0, 0)),
            pl.BlockSpec((GD2, OUT), lambda i: (0, 0)),
            pl.BlockSpec((1, OUT), lambda i: (0, 0)),
        ],
        out_specs=pl.BlockSpec((G, OUT), lambda i: (0, 0)),
        out_shape=jax.ShapeDtypeStruct((G, OUT), jnp.float32),
        scratch_shapes=[pltpu.VMEM((G, GD), jnp.float32)],
    )(agg_parts, hp3, dinv, b3.reshape(1, DIM), x1, x2, batch3d,
      fc1_W, fc1_b.reshape(1, FD), fc2a_W, fc2a_b.reshape(1, GD),
      fc2b_W, fc2b_b.reshape(1, GD2), fc3_W, fc3_b.reshape(1, OUT))


def kernel(x, edge_index, batch, W1, b1, W2, b2, W3, b3,
           fc1_W, fc1_b, fc2a_W, fc2a_b, fc2_W, fc2_b,
           fc2b_W, fc2b_b, fc3_W, fc3_b):
    del fc2_W, fc2_b

    pad = E_PAD - E
    src = jnp.concatenate([edge_index[0], jnp.zeros((pad,), jnp.int32)]).reshape(NW, J, C)
    dst = jnp.concatenate([edge_index[1], jnp.full((pad,), N, jnp.int32)]).reshape(NW, J, C)

    ones_deg = jnp.ones((C, DEGW), jnp.float32)
    zeros_deg = jnp.zeros((NACC, DEGW), jnp.float32)
    zeros_acc = jnp.zeros((NACC, DIM), jnp.float32)

    h1 = _tc_mm1(x, W1)
    deg_parts = _sc_degree(dst, ones_deg, zeros_deg)
    hp1, dinv = _tc_scale(deg_parts, h1)

    agg1 = _sc_edge_agg(src, dst, hp1, zeros_acc)
    x1, hp2 = _tc_mid(agg1, hp1, dinv, b1, W2)

    agg2 = _sc_edge_agg(src, dst, hp2, zeros_acc)
    x2, hp3 = _tc_mid(agg2, hp2, dinv, b2, W3)

    agg3 = _sc_edge_agg(src, dst, hp3, zeros_acc)

    batch3d = batch.reshape(_GRID, 1, _R)
    return _tc_head(agg3, hp3, dinv, b3, x1, x2, batch3d,
                    fc1_W, fc1_b, fc2a_W, fc2a_b, fc2b_W, fc2b_b, fc3_W, fc3_b)


# speedup vs baseline: 24.6507x; 1.0146x over previous
import functools

import jax
import jax.numpy as jnp
from jax import lax
from jax.experimental import pallas as pl
from jax.experimental.pallas import tpu as pltpu
from jax.experimental.pallas import tpu_sc as plsc

N = 10000
E = 320000
G = 64
D_IN = 128
DIM = 32

NC = 2
NS = 16
NW = NC * NS
C = 128
J = 80
E_PAD = NW * J * C
NACC = 10112
ZR = NACC // NS
DEGW = 8
NBUF = 4

_mesh = plsc.VectorSubcoreMesh(core_axis_name="c", subcore_axis_name="s")
_sc_params = pltpu.CompilerParams(use_tc_tiling_on_sc=False)


@functools.partial(
    pl.kernel,
    out_type=jax.ShapeDtypeStruct((NC, NACC, DEGW), jnp.float32),
    mesh=_mesh,
    scratch_types=(
        [pltpu.VMEM((J, C), jnp.int32),
         pltpu.VMEM((C, DEGW), jnp.float32),
         pltpu.VMEM_SHARED((NACC, DEGW), jnp.float32)]
        + [pltpu.SemaphoreType.DMA] * 4
    ),
    compiler_params=_sc_params,
)
def _sc_degree(dst_hbm, ones_hbm, zeros_hbm, out_hbm, didx, ones_v, acc, *ssem):
    cid = lax.axis_index("c")
    sid = lax.axis_index("s")
    wid = sid * NC + cid
    pltpu.sync_copy(ones_hbm, ones_v)
    pltpu.sync_copy(dst_hbm.at[wid], didx)
    pltpu.sync_copy(zeros_hbm.at[pl.ds(sid * ZR, ZR)], acc.at[pl.ds(sid * ZR, ZR)])
    plsc.subcore_barrier()

    def S(j, m):
        pltpu.async_copy(ones_v, acc.at[didx.at[j]], ssem[m], add=True)

    def Sw(j, m):
        pltpu.make_async_copy(ones_v, acc.at[didx.at[j]], ssem[m]).wait()

    def body(jj, carry):
        for b in range(4):
            j = jj * 4 + b
            S(j, b)

            @pl.when(j >= 2)
            def _drain():
                Sw(j - 2, (b + 2) % 4)

        return carry

    lax.fori_loop(0, J // 4, body, 0)
    Sw(J - 2, (J - 2) % 4)
    Sw(J - 1, (J - 1) % 4)
    plsc.subcore_barrier()
    pltpu.sync_copy(acc.at[pl.ds(sid * ZR, ZR)], out_hbm.at[cid, pl.ds(sid * ZR, ZR)])


_NR = 4
_GL = 2


@functools.partial(
    pl.kernel,
    out_type=jax.ShapeDtypeStruct((NC, NACC, DIM), jnp.float32),
    mesh=_mesh,
    scratch_types=(
        [pltpu.VMEM((J, C), jnp.int32)] * 2
        + [pltpu.VMEM((C, DIM), jnp.float32)] * _NR
        + [pltpu.VMEM_SHARED((NACC, DIM), jnp.float32)]
        + [pltpu.SemaphoreType.DMA] * (2 * _NR)
    ),
    compiler_params=_sc_params,
)
def _sc_edge_agg(src_hbm, dst_hbm, hp_hbm, zeros_hbm, out_hbm, *scr):
    sidx = scr[0]
    didx = scr[1]
    rows = scr[2:2 + _NR]
    acc = scr[2 + _NR]
    sems = scr[3 + _NR:]
    gsem = sems[:_NR]
    ssem = sems[_NR:]
    cid = lax.axis_index("c")
    sid = lax.axis_index("s")
    wid = sid * NC + cid
    pltpu.sync_copy(src_hbm.at[wid], sidx)
    pltpu.sync_copy(dst_hbm.at[wid], didx)
    pltpu.sync_copy(zeros_hbm.at[pl.ds(sid * ZR, ZR)], acc.at[pl.ds(sid * ZR, ZR)])
    plsc.subcore_barrier()

    def G(j, r):
        pltpu.async_copy(hp_hbm.at[sidx.at[j]], rows[r], gsem[r])

    def Gw(j, r):
        pltpu.make_async_copy(hp_hbm.at[sidx.at[j]], rows[r], gsem[r]).wait()

    def S(j, r):
        pltpu.async_copy(rows[r], acc.at[didx.at[j]], ssem[r], add=True)

    def Sw(j, r):
        pltpu.make_async_copy(rows[r], acc.at[didx.at[j]], ssem[r]).wait()

    for m in range(_GL):
        G(m, m)

    def body(jj, carry):
        for b in range(_NR):
            j = jj * _NR + b
            Gw(j, b)
            S(j, b)

            @pl.when(j >= _GL)
            def _drain():
                Sw(j - _GL, (b + _NR - _GL) % _NR)

            @pl.when(j + _GL < J)
            def _gather_ahead():
                G(j + _GL, (b + _GL) % _NR)

        return carry

    lax.fori_loop(0, J // _NR, body, 0)
    for k in range(J - _GL, J):
        Sw(k, k % _NR)
    plsc.subcore_barrier()
    pltpu.sync_copy(acc.at[pl.ds(sid * ZR, ZR)], out_hbm.at[cid, pl.ds(sid * ZR, ZR)])


_R = 1000
_GRID = N // _R


def _mm1_body(x_ref, w1_ref, h_ref):
    h_ref[...] = jnp.dot(x_ref[...], w1_ref[...], preferred_element_type=jnp.float32)


def _tc_mm1(x, W1):
    return pl.pallas_call(
        _mm1_body,
        grid=(_GRID,),
        in_specs=[
            pl.BlockSpec((_R, D_IN), lambda i: (i, 0)),
            pl.BlockSpec((D_IN, DIM), lambda i: (0, 0)),
        ],
        out_specs=pl.BlockSpec((_R, DIM), lambda i: (i, 0)),
        out_shape=jax.ShapeDtypeStruct((N, DIM), jnp.float32),
    )(x, W1)


def _scale_body(deg_ref, h_ref, hp_ref, dinv_ref):
    deg = deg_ref[0, :, 0:1] + deg_ref[1, :, 0:1] + 1.0
    dinv = lax.rsqrt(deg)
    hp_ref[...] = h_ref[...] * dinv
    dinv_ref[...] = dinv


def _tc_scale(deg_parts, h1):
    return pl.pallas_call(
        _scale_body,
        grid=(_GRID,),
        in_specs=[
            pl.BlockSpec((NC, _R, DEGW), lambda i: (0, i, 0)),
            pl.BlockSpec((_R, DIM), lambda i: (i, 0)),
        ],
        out_specs=[
            pl.BlockSpec((_R, DIM), lambda i: (i, 0)),
            pl.BlockSpec((_R, 1), lambda i: (i, 0)),
        ],
        out_shape=[
            jax.ShapeDtypeStruct((N, DIM), jnp.float32),
            jax.ShapeDtypeStruct((N, 1), jnp.float32),
        ],
    )(deg_parts, h1)


def _mid_body(agg_ref, hp_ref, dinv_ref, b_ref, w_ref, x_ref, hpn_ref):
    dinv = dinv_ref[...]
    t = (agg_ref[0] + agg_ref[1] + hp_ref[...]) * dinv + b_ref[...]
    xi = jnp.maximum(t, 0.0)
    x_ref[...] = xi
    hpn_ref[...] = jnp.dot(xi, w_ref[...], preferred_element_type=jnp.float32) * dinv


def _tc_mid(agg_parts, hp, dinv, b, Wn):
    return pl.pallas_call(
        _mid_body,
        grid=(_GRID,),
        in_specs=[
            pl.BlockSpec((NC, _R, DIM), lambda i: (0, i, 0)),
            pl.BlockSpec((_R, DIM), lambda i: (i, 0)),
            pl.BlockSpec((_R, 1), lambda i: (i, 0)),
            pl.BlockSpec((1, DIM), lambda i: (0, 0)),
            pl.BlockSpec((DIM, DIM), lambda i: (0, 0)),
        ],
        out_specs=[
            pl.BlockSpec((_R, DIM), lambda i: (i, 0)),
            pl.BlockSpec((_R, DIM), lambda i: (i, 0)),
        ],
        out_shape=[
            jax.ShapeDtypeStruct((N, DIM), jnp.float32),
            jax.ShapeDtypeStruct((N, DIM), jnp.float32),
        ],
    )(agg_parts, hp, dinv, b.reshape(1, DIM), Wn)


def _head_body(agg_ref, hp3_ref, dinv_ref, b3_ref, x1_ref, x2_ref, batch_ref,
               fc1w_ref, fc1b_ref, fc2aw_ref, fc2ab_ref,
               fc2bw_ref, fc2bb_ref, fc3w_ref, fc3b_ref,
               out_ref, pooled_scr):
    i = pl.program_id(0)

    @pl.when(i == 0)
    def _zero():
        pooled_scr[...] = jnp.zeros_like(pooled_scr)

    dinv = dinv_ref[...]
    t = (agg_ref[0] + agg_ref[1] + hp3_ref[...]) * dinv + b3_ref[...]
    x3 = jnp.maximum(t, 0.0)
    h = (jnp.dot(x1_ref[...], fc1w_ref[0:DIM], preferred_element_type=jnp.float32)
         + jnp.dot(x2_ref[...], fc1w_ref[DIM:2 * DIM], preferred_element_type=jnp.float32)
         + jnp.dot(x3, fc1w_ref[2 * DIM:3 * DIM], preferred_element_type=jnp.float32)
         + fc1b_ref[...])
    h = jnp.maximum(h, 0.0)
    h2 = jnp.dot(h, fc2aw_ref[...], preferred_element_type=jnp.float32) + fc2ab_ref[...]
    h2 = jnp.maximum(h2, 0.0)
    gids = lax.broadcasted_iota(jnp.int32, (G, _R), 0)
    onehot = (gids == batch_ref[0]).astype(jnp.float32)
    pooled_scr[...] += jnp.dot(onehot, h2, preferred_element_type=jnp.float32)

    @pl.when(i == pl.num_programs(0) - 1)
    def _head():
        ph = pooled_scr[...]
        hb = jnp.dot(ph, fc2bw_ref[...], preferred_element_type=jnp.float32) + fc2bb_ref[...]
        hb = jnp.maximum(hb, 0.0)
        lg = jnp.dot(hb, fc3w_ref[...], preferred_element_type=jnp.float32) + fc3b_ref[...]
        m = jnp.max(lg, axis=-1, keepdims=True)
        s = jnp.sum(jnp.exp(lg - m), axis=-1, keepdims=True)
        out_ref[...] = (lg - m) - jnp.log(s)


def _tc_head(agg_parts, hp3, dinv, b3, x1, x2, batch3d,
             fc1_W, fc1_b, fc2a_W, fc2a_b, fc2b_W, fc2b_b, fc3_W, fc3_b):
    OUT = fc3_W.shape[1]
    GD2 = fc2b_W.shape[1]
    GD = fc2a_W.shape[1]
    FD = fc1_W.shape[1]
    return pl.pallas_call(
        _head_body,
        grid=(_GRID,),
        in_specs=[
            pl.BlockSpec((NC, _R, DIM), lambda i: (0, i, 0)),
            pl.BlockSpec((_R, DIM), lambda i: (i, 0)),
            pl.BlockSpec((_R, 1), lambda i: (i, 0)),
            pl.BlockSpec((1, DIM), lambda i: (0, 0)),
            pl.BlockSpec((_R, DIM), lambda i: (i, 0)),
            pl.BlockSpec((_R, DIM), lambda i: (i, 0)),
            pl.BlockSpec((1, 1, _R), lambda i: (i, 0, 0)),
            pl.BlockSpec((3 * DIM, FD), lambda i: (0, 0)),
            pl.BlockSpec((1, FD), lambda i: (0, 0)),
            pl.BlockSpec((FD, GD), lambda i: (0, 0)),
            pl.BlockSpec((1, GD), lambda i: (0, 0)),
            pl.BlockSpec((GD, GD2), lambda i: (0, 0)),
            pl.BlockSpec((1, GD2), lambda i: (0, 0)),
            pl.BlockSpec((GD2, OUT), lambda i: (0, 0)),
            pl.BlockSpec((1, OUT), lambda i: (0, 0)),
        ],
        out_specs=pl.BlockSpec((G, OUT), lambda i: (0, 0)),
        out_shape=jax.ShapeDtypeStruct((G, OUT), jnp.float32),
        scratch_shapes=[pltpu.VMEM((G, GD), jnp.float32)],
    )(agg_parts, hp3, dinv, b3.reshape(1, DIM), x1, x2, batch3d,
      fc1_W, fc1_b.reshape(1, FD), fc2a_W, fc2a_b.reshape(1, GD),
      fc2b_W, fc2b_b.reshape(1, GD2), fc3_W, fc3_b.reshape(1, OUT))


def kernel(x, edge_index, batch, W1, b1, W2, b2, W3, b3,
           fc1_W, fc1_b, fc2a_W, fc2a_b, fc2_W, fc2_b,
           fc2b_W, fc2b_b, fc3_W, fc3_b):
    del fc2_W, fc2_b

    pad = E_PAD - E
    src = jnp.concatenate([edge_index[0], jnp.zeros((pad,), jnp.int32)]).reshape(NW, J, C)
    dst = jnp.concatenate([edge_index[1], jnp.full((pad,), N, jnp.int32)]).reshape(NW, J, C)

    ones_deg = jnp.ones((C, DEGW), jnp.float32)
    zeros_deg = jnp.zeros((NACC, DEGW), jnp.float32)
    zeros_acc = jnp.zeros((NACC, DIM), jnp.float32)

    h1 = _tc_mm1(x, W1)
    deg_parts = _sc_degree(dst, ones_deg, zeros_deg)
    hp1, dinv = _tc_scale(deg_parts, h1)

    agg1 = _sc_edge_agg(src, dst, hp1, zeros_acc)
    x1, hp2 = _tc_mid(agg1, hp1, dinv, b1, W2)

    agg2 = _sc_edge_agg(src, dst, hp2, zeros_acc)
    x2, hp3 = _tc_mid(agg2, hp2, dinv, b2, W3)

    agg3 = _sc_edge_agg(src, dst, hp3, zeros_acc)

    batch3d = batch.reshape(_GRID, 1, _R)
    return _tc_head(agg3, hp3, dinv, b3, x1, x2, batch3d,
                    fc1_W, fc1_b, fc2a_W, fc2a_b, fc2b_W, fc2b_b, fc3_W, fc3_b)
